# Initial kernel scaffold; baseline (speedup 1.0000x reference)
import jax
import jax.numpy as jnp
from jax.experimental import pallas as pl


def kernel(x, adj, W_e1, b_e1, W_e2, b_e2, W_a1, b_a1, W_a2, b_a2, W_s1, b_s1):
    raise NotImplementedError("write your pallas kernel here")


# speedup vs baseline: 9.0191x; 9.0191x over previous
import functools

import jax
import jax.numpy as jnp
from jax import lax
from jax.experimental import pallas as pl
from jax.experimental.pallas import tpu as pltpu
from jax.experimental.pallas import tpu_sc as plsc

N = 10000
E = 320000
BN = 1000
GRID = N // BN

K = 80
ROWS = E // K
NW = 32
CPW = ROWS // NW
ZR = 40
NG = N // ZR


def _proj_rows(z):
    space_sq = jnp.sum(z * z, axis=1, keepdims=True) - z[:, :1] ** 2
    time = jnp.sqrt(1.0 + space_sq)
    col = lax.broadcasted_iota(jnp.int32, z.shape, 1)
    return jnp.where(col == 0, time, z)


def _centralize_rows(a):
    sum_sq = jnp.sum(a * a, axis=1, keepdims=True)
    inner = sum_sq - 2.0 * a[:, :1] ** 2
    denom = jnp.sqrt(jnp.clip(-inner, 1e-8, None))
    return a / denom


def _linear_proj(x, w_ref, b_ref):
    z = lax.dot_general(x, w_ref[...], (((1,), (1,)), ((), ())),
                        preferred_element_type=jnp.float32) + b_ref[...]
    return _proj_rows(z)


def _combine(p_ref, deg):
    s = p_ref[0] + p_ref[1]
    return s / deg


def _tc1_body(x_ref, w_ref, b_ref, o_ref):
    h = _linear_proj(x_ref[...], w_ref, b_ref)
    padcol = lax.broadcasted_iota(jnp.int32, (h.shape[0], 16), 1)
    pad = jnp.where(padcol == 0, 1.0, 0.0).astype(jnp.float32)
    o_ref[...] = jnp.concatenate([h, pad], axis=1)


def _tc2_body(p_ref, w_ref, b_ref, h_ref, deg_ref):
    s = p_ref[0] + p_ref[1]
    deg = jnp.clip(s[:, 64:65], 1.0, None)
    c = _centralize_rows(s[:, :64] / deg)
    h_ref[...] = _linear_proj(c, w_ref, b_ref)
    deg_ref[...] = jnp.broadcast_to(deg, (deg.shape[0], 8))


def _tc3_body(p_ref, deg_ref, w3_ref, b3_ref, w5_ref, b5_ref, o_ref):
    deg = deg_ref[...][:, :1]
    xe = _centralize_rows(_combine(p_ref, deg))
    h3 = _linear_proj(xe, w3_ref, b3_ref)
    h5 = _linear_proj(xe, w5_ref, b5_ref)
    o_ref[...] = jnp.concatenate([h3, h5], axis=1)


def _tc4_body(p_ref, deg_ref, w_ref, b_ref, h_ref, struct_ref):
    deg = deg_ref[...][:, :1]
    s = p_ref[0] + p_ref[1]
    h2c = _centralize_rows(s[:, :64] / deg)
    h_ref[...] = _linear_proj(h2c, w_ref, b_ref)
    struct_ref[...] = _centralize_rows(s[:, 64:96] / deg)


def _tc5_body(p_ref, deg_ref, o_ref):
    deg = deg_ref[...][:, :1]
    o_ref[...] = _centralize_rows(_combine(p_ref, deg))


def _wspec(shape):
    nd = len(shape)
    return pl.BlockSpec(shape, lambda i: (0,) * nd)


def _pspec(d):
    return pl.BlockSpec((2, BN, d), lambda i: (0, i, 0))


def _nspec(d):
    return pl.BlockSpec((BN, d), lambda i: (i, 0))


def _tc_call(body, in_specs, out_specs, out_shape):
    return pl.pallas_call(body, grid=(GRID,), in_specs=in_specs,
                          out_specs=out_specs, out_shape=out_shape)


def _sc_agg(table, src_r, dst_r, d):
    mesh = plsc.VectorSubcoreMesh(core_axis_name="c", subcore_axis_name="s")

    @functools.partial(
        pl.kernel,
        mesh=mesh,
        compiler_params=pltpu.CompilerParams(use_tc_tiling_on_sc=False),
        out_type=jax.ShapeDtypeStruct((2, N, d), jnp.float32),
        scratch_types=[
            pltpu.VMEM((CPW, K), jnp.int32),
            pltpu.VMEM((CPW, K), jnp.int32),
            pltpu.VMEM((K, d), jnp.float32),
            pltpu.VMEM((ZR, d), jnp.float32),
            pltpu.VMEM_SHARED((N, d), jnp.float32),
            pltpu.SemaphoreType.DMA,
        ],
    )
    def k(table_hbm, src_hbm, dst_hbm, out_hbm,
          src_v, dst_v, rows_v, zbuf, acc, sem):
        cid = lax.axis_index("c")
        sid = lax.axis_index("s")
        wid = sid * 2 + cid

        zero16 = jnp.zeros((16,), jnp.float32)
        for r in range(ZR):
            for j in range(d // 16):
                zbuf[r, pl.ds(j * 16, 16)] = zero16

        def zrow(i, carry):
            g = pl.multiple_of((i * 16 + sid) * ZR, 8)
            pltpu.sync_copy(zbuf, acc.at[pl.ds(g, ZR)])
            return carry
        lax.fori_loop(0, NG // 16, zrow, 0)

        @pl.when(sid < NG % 16)
        def _():
            g = pl.multiple_of(((NG // 16) * 16 + sid) * ZR, 8)
            pltpu.sync_copy(zbuf, acc.at[pl.ds(g, ZR)])

        plsc.subcore_barrier()

        pltpu.sync_copy(src_hbm.at[wid], src_v)
        pltpu.sync_copy(dst_hbm.at[wid], dst_v)

        def body(j, carry):
            pltpu.async_copy(table_hbm.at[src_v.at[j]], rows_v, sem).wait()
            pltpu.sync_copy(rows_v, acc.at[dst_v.at[j]], add=True)
            return carry
        lax.fori_loop(0, CPW, body, 0)
        plsc.subcore_barrier()

        def drow(i, carry):
            g = pl.multiple_of((i * 16 + sid) * ZR, 8)
            pltpu.sync_copy(acc.at[pl.ds(g, ZR)], out_hbm.at[cid, pl.ds(g, ZR)])
            return carry
        lax.fori_loop(0, NG // 16, drow, 0)

        @pl.when(sid < NG % 16)
        def _():
            g = pl.multiple_of(((NG // 16) * 16 + sid) * ZR, 8)
            pltpu.sync_copy(acc.at[pl.ds(g, ZR)], out_hbm.at[cid, pl.ds(g, ZR)])

    return k(table, src_r, dst_r)


def kernel(x, adj, W_e1, b_e1, W_e2, b_e2, W_a1, b_a1, W_a2, b_a2, W_s1, b_s1):
    src_r = adj[0].reshape(NW, CPW, K)
    dst_r = adj[1].reshape(NW, CPW, K)
    b_e1 = b_e1.reshape(1, -1)
    b_e2 = b_e2.reshape(1, -1)
    b_a1 = b_a1.reshape(1, -1)
    b_a2 = b_a2.reshape(1, -1)
    b_s1 = b_s1.reshape(1, -1)

    h1 = _tc_call(
        _tc1_body,
        [_nspec(128), _wspec((64, 128)), _wspec((1, 64))],
        _nspec(80),
        jax.ShapeDtypeStruct((N, 80), jnp.float32),
    )(x, W_e1, b_e1)

    p1 = _sc_agg(h1, src_r, dst_r, 80)

    h2, deg8 = _tc_call(
        _tc2_body,
        [_pspec(80), _wspec((32, 64)), _wspec((1, 32))],
        [_nspec(32), _nspec(8)],
        (jax.ShapeDtypeStruct((N, 32), jnp.float32),
         jax.ShapeDtypeStruct((N, 8), jnp.float32)),
    )(p1, W_e2, b_e2)

    p2 = _sc_agg(h2, src_r, dst_r, 32)

    h35 = _tc_call(
        _tc3_body,
        [_pspec(32), _nspec(8), _wspec((64, 32)), _wspec((1, 64)),
         _wspec((32, 32)), _wspec((1, 32))],
        _nspec(96),
        jax.ShapeDtypeStruct((N, 96), jnp.float32),
    )(p2, deg8, W_a1, b_a1, W_s1, b_s1)

    p35 = _sc_agg(h35, src_r, dst_r, 96)

    h4, struct = _tc_call(
        _tc4_body,
        [_pspec(96), _nspec(8), _wspec((128, 64)), _wspec((1, 128))],
        [_nspec(128), _nspec(32)],
        (jax.ShapeDtypeStruct((N, 128), jnp.float32),
         jax.ShapeDtypeStruct((N, 32), jnp.float32)),
    )(p35, deg8, W_a2, b_a2)

    p4 = _sc_agg(h4, src_r, dst_r, 128)

    x_hat = _tc_call(
        _tc5_body,
        [_pspec(128), _nspec(8)],
        _nspec(128),
        jax.ShapeDtypeStruct((N, 128), jnp.float32),
    )(p4, deg8)

    return (x_hat, struct)
